# baseline (speedup 1.0000x reference)
import functools

import jax
import jax.numpy as jnp
from jax import lax
from jax.experimental import pallas as pl
from jax.experimental.pallas import tpu as pltpu
from jax.experimental.pallas import tpu_sc as plsc

S, H, I, E = 2048, 768, 3072, 64
C = 64
NCHUNK = S // C + E
P = NCHUNK * C
EPS = 1e-12
NW = 32
RTB = 256
DTB = 512


def _route_body(x_ref, wr_ref, pos_ref, startc_ref, nch_ref):
    x = x_ref[...]
    wr = wr_ref[...]
    logits = lax.dot_general(x, wr, (((1,), (1,)), ((), ())),
                             preferred_element_type=jnp.float32)
    row_max = jnp.max(logits, axis=1, keepdims=True)
    eiota = lax.broadcasted_iota(jnp.int32, (S, E), 1)
    eid = jnp.min(jnp.where(logits >= row_max, eiota, E), axis=1, keepdims=True)
    onehot = (eid == eiota).astype(jnp.float32)

    counts = jnp.sum(onehot, axis=0, keepdims=True)
    pc = jnp.ceil(counts * (1.0 / C)) * C
    ej = lax.broadcasted_iota(jnp.int32, (E, E), 0)
    ek = lax.broadcasted_iota(jnp.int32, (E, E), 1)
    strict_lt = (ej < ek).astype(jnp.float32)
    po = jnp.dot(pc, strict_lt, preferred_element_type=jnp.float32)

    tj = lax.broadcasted_iota(jnp.int32, (RTB, RTB), 0)
    tk = lax.broadcasted_iota(jnp.int32, (RTB, RTB), 1)
    tril = (tk < tj).astype(jnp.float32)
    running = jnp.zeros((1, E), jnp.float32)
    for b in range(S // RTB):
        oh = onehot[b * RTB:(b + 1) * RTB, :]
        prev = jnp.dot(tril, oh, preferred_element_type=jnp.float32) + running
        dest = jnp.sum((prev + po) * oh, axis=1, keepdims=True)
        pos_ref[b * RTB:(b + 1) * RTB, :] = dest.astype(jnp.int32)
        running = running + jnp.sum(oh, axis=0, keepdims=True)

    eyeE = (ej == ek).astype(jnp.float32)
    pc_col = lax.dot_general(eyeE, pc, (((1,), (1,)), ((), ())))
    po_col = jnp.dot((ek < ej).astype(jnp.float32), pc_col,
                     preferred_element_type=jnp.float32)
    startc_ref[...] = (po_col * (1.0 / C)).astype(jnp.int32)
    nch_ref[...] = (pc_col * (1.0 / C)).astype(jnp.int32)


def _routing(x, w_router):
    return pl.pallas_call(
        _route_body,
        out_shape=(jax.ShapeDtypeStruct((S, 1), jnp.int32),
                   jax.ShapeDtypeStruct((E, 1), jnp.int32),
                   jax.ShapeDtypeStruct((E, 1), jnp.int32)),
    )(x, w_router)


def _sc_wid():
    return lax.axis_index("s") * 2 + lax.axis_index("c")


def _scatter_tokens(x, pos):
    tpw = S // NW
    mesh = plsc.VectorSubcoreMesh(core_axis_name="c", subcore_axis_name="s")

    @functools.partial(
        pl.kernel, mesh=mesh,
        out_type=jax.ShapeDtypeStruct((P, H), jnp.float32),
        scratch_types=[pltpu.VMEM((tpw,), jnp.int32),
                       pltpu.VMEM((tpw, H), jnp.float32)],
    )
    def k(x_hbm, pos_hbm, out_hbm, idx_v, rows_v):
        base = _sc_wid() * tpw
        pltpu.sync_copy(pos_hbm.at[pl.ds(base, tpw)], idx_v)
        pltpu.sync_copy(x_hbm.at[pl.ds(base, tpw)], rows_v)
        pltpu.sync_copy(rows_v, out_hbm.at[idx_v])

    return k(x, pos)


def _gather_inter(up, pos):
    tpw = S // NW
    sub = 16
    nsub = tpw // sub
    mesh = plsc.VectorSubcoreMesh(core_axis_name="c", subcore_axis_name="s")

    @functools.partial(
        pl.kernel, mesh=mesh,
        out_type=jax.ShapeDtypeStruct((S, I), jnp.float32),
        scratch_types=[pltpu.VMEM((sub,), jnp.int32),
                       pltpu.VMEM((sub, I), jnp.float32),
                       pltpu.VMEM((sub, I), jnp.float32),
                       pltpu.SemaphoreType.DMA((2,))],
    )
    def k(up_hbm, pos_hbm, out_hbm, idx_v, r0, r1, wsems):
        wbase = _sc_wid() * tpw
        rows = [r0, r1]

        def wb(j, b):
            return pltpu.make_async_copy(
                rows[b], out_hbm.at[pl.ds(wbase + j * sub, sub)], wsems.at[b])

        for j in range(nsub):
            b = j % 2
            if j >= 2:
                wb(j - 2, b).wait()
            pltpu.sync_copy(pos_hbm.at[pl.ds(wbase + j * sub, sub)], idx_v)
            pltpu.sync_copy(up_hbm.at[idx_v], rows[b])
            wb(j, b).start()
        wb(nsub - 2, 0).wait()
        wb(nsub - 1, 1).wait()

    return k(up, pos)


def _gelu_exact(h):
    return 0.5 * h * (1.0 + lax.erf(h * 0.7071067811865476))


def _up_body(sc_ref, nc_ref, xs_ref, wi_ref, bi_ref, out_hbm, stg0, stg1, sems):
    e = pl.program_id(0)
    start = sc_ref[e]
    n = nc_ref[e]

    def dma(k, stg, idx):
        return pltpu.make_async_copy(
            stg, out_hbm.at[pl.ds((start + k) * C, C), :], sems.at[idx])

    def compute(k, stg):
        row = (start + k) * C
        xb = xs_ref[pl.ds(row, C), :].astype(jnp.bfloat16)
        h = jnp.dot(xb, wi_ref[0].astype(jnp.bfloat16),
                    preferred_element_type=jnp.float32)
        stg[...] = _gelu_exact(h + bi_ref[0])

    def pair(k2, carry):
        k = k2 * 2

        @pl.when(k2 > 0)
        def _():
            dma(0, stg0, 0).wait()

        compute(k, stg0)
        dma(k, stg0, 0).start()

        @pl.when(k + 1 < n)
        def _():
            @pl.when(k2 > 0)
            def _():
                dma(0, stg1, 1).wait()

            compute(k + 1, stg1)
            dma(k + 1, stg1, 1).start()

        return carry

    lax.fori_loop(0, (n + 1) // 2, pair, 0)

    @pl.when(n > 0)
    def _():
        dma(0, stg0, 0).wait()

    @pl.when(n > 1)
    def _():
        dma(0, stg1, 1).wait()


def _up(startc, nch, x_sorted, wi, bi):
    grid_spec = pltpu.PrefetchScalarGridSpec(
        num_scalar_prefetch=2,
        grid=(E,),
        in_specs=[
            pl.BlockSpec((P, H), lambda e, sc, nc: (0, 0)),
            pl.BlockSpec((1, H, I), lambda e, sc, nc: (e, 0, 0)),
            pl.BlockSpec((1, 1, I), lambda e, sc, nc: (e, 0, 0)),
        ],
        out_specs=pl.BlockSpec(memory_space=pl.ANY),
        scratch_shapes=[
            pltpu.VMEM((C, I), jnp.float32),
            pltpu.VMEM((C, I), jnp.float32),
            pltpu.SemaphoreType.DMA((2,)),
        ],
    )
    return pl.pallas_call(
        _up_body, grid_spec=grid_spec,
        out_shape=jax.ShapeDtypeStruct((P, I), jnp.float32),
    )(startc, nch, x_sorted, wi, bi.reshape(E, 1, I))


def _down_body(inter_ref, wo_ref, bo_ref, x_ref, g_ref, b_ref, y_ref):
    o = jnp.dot(inter_ref[...].astype(jnp.bfloat16),
                wo_ref[...].astype(jnp.bfloat16),
                preferred_element_type=jnp.float32)
    t = o + bo_ref[...] + x_ref[...]
    mu = jnp.mean(t, axis=1, keepdims=True)
    cen = t - mu
    var = jnp.mean(cen * cen, axis=1, keepdims=True)
    y_ref[...] = cen * lax.rsqrt(var + EPS) * g_ref[...] + b_ref[...]


def _down(inter, wo, bo, x, ln_g, ln_b):
    return pl.pallas_call(
        _down_body,
        grid=(S // DTB,),
        in_specs=[
            pl.BlockSpec((DTB, I), lambda t: (t, 0)),
            pl.BlockSpec((I, H), lambda t: (0, 0)),
            pl.BlockSpec((1, H), lambda t: (0, 0)),
            pl.BlockSpec((DTB, H), lambda t: (t, 0)),
            pl.BlockSpec((1, H), lambda t: (0, 0)),
            pl.BlockSpec((1, H), lambda t: (0, 0)),
        ],
        out_specs=pl.BlockSpec((DTB, H), lambda t: (t, 0)),
        out_shape=jax.ShapeDtypeStruct((S, H), jnp.float32),
    )(inter, wo, bo, x, ln_g, ln_b)


def kernel(hidden_states, w_router, wi, bi, wo, bo, ln_g, ln_b):
    b, s, h = hidden_states.shape
    x = hidden_states.reshape(s, h)
    pos2, startc2, nch2 = _routing(x, w_router)
    pos = pos2.reshape(s)
    x_sorted = _scatter_tokens(x, pos)
    up = _up(startc2.reshape(E), nch2.reshape(E), x_sorted, wi, bi)
    inter = _gather_inter(up, pos)
    y = _down(inter, wo, bo.reshape(1, h), x, ln_g.reshape(1, h), ln_b.reshape(1, h))
    return y.reshape(b, s, h)


# speedup vs baseline: 1.0118x; 1.0118x over previous
import functools

import jax
import jax.numpy as jnp
from jax import lax
from jax.experimental import pallas as pl
from jax.experimental.pallas import tpu as pltpu
from jax.experimental.pallas import tpu_sc as plsc

S, H, I, E = 2048, 768, 3072, 64
C = 64
NCHUNK = S // C + E
P = NCHUNK * C
EPS = 1e-12
NW = 32
RTB = 256
DTB = 512


def _route_body(x_ref, wr_ref, pos_ref, startc_ref, nch_ref):
    x = x_ref[...]
    wr = wr_ref[...]
    logits = lax.dot_general(x, wr, (((1,), (1,)), ((), ())),
                             preferred_element_type=jnp.float32)
    row_max = jnp.max(logits, axis=1, keepdims=True)
    eiota = lax.broadcasted_iota(jnp.int32, (S, E), 1)
    eid = jnp.min(jnp.where(logits >= row_max, eiota, E), axis=1, keepdims=True)
    onehot = (eid == eiota).astype(jnp.float32)

    counts = jnp.sum(onehot, axis=0, keepdims=True)
    pc = jnp.ceil(counts * (1.0 / C)) * C
    ej = lax.broadcasted_iota(jnp.int32, (E, E), 0)
    ek = lax.broadcasted_iota(jnp.int32, (E, E), 1)
    strict_lt = (ej < ek).astype(jnp.float32)
    po = jnp.dot(pc, strict_lt, preferred_element_type=jnp.float32)

    tj = lax.broadcasted_iota(jnp.int32, (RTB, RTB), 0)
    tk = lax.broadcasted_iota(jnp.int32, (RTB, RTB), 1)
    tril = (tk < tj).astype(jnp.float32)
    running = jnp.zeros((1, E), jnp.float32)
    for b in range(S // RTB):
        oh = onehot[b * RTB:(b + 1) * RTB, :]
        prev = jnp.dot(tril, oh, preferred_element_type=jnp.float32) + running
        dest = jnp.sum((prev + po) * oh, axis=1, keepdims=True)
        pos_ref[b * RTB:(b + 1) * RTB, :] = dest.astype(jnp.int32)
        running = running + jnp.sum(oh, axis=0, keepdims=True)

    eyeE = (ej == ek).astype(jnp.float32)
    pc_col = lax.dot_general(eyeE, pc, (((1,), (1,)), ((), ())))
    po_col = jnp.dot((ek < ej).astype(jnp.float32), pc_col,
                     preferred_element_type=jnp.float32)
    startc_ref[...] = (po_col * (1.0 / C)).astype(jnp.int32)
    nch_ref[...] = (pc_col * (1.0 / C)).astype(jnp.int32)


def _routing(x, w_router):
    return pl.pallas_call(
        _route_body,
        out_shape=(jax.ShapeDtypeStruct((S, 1), jnp.int32),
                   jax.ShapeDtypeStruct((E, 1), jnp.int32),
                   jax.ShapeDtypeStruct((E, 1), jnp.int32)),
    )(x, w_router)


def _sc_wid():
    return lax.axis_index("s") * 2 + lax.axis_index("c")


def _scatter_tokens(x, pos):
    tpw = S // NW
    mesh = plsc.VectorSubcoreMesh(core_axis_name="c", subcore_axis_name="s")

    @functools.partial(
        pl.kernel, mesh=mesh,
        out_type=jax.ShapeDtypeStruct((P, H), jnp.float32),
        scratch_types=[pltpu.VMEM((tpw,), jnp.int32),
                       pltpu.VMEM((tpw, H), jnp.float32)],
    )
    def k(x_hbm, pos_hbm, out_hbm, idx_v, rows_v):
        base = _sc_wid() * tpw
        pltpu.sync_copy(pos_hbm.at[pl.ds(base, tpw)], idx_v)
        pltpu.sync_copy(x_hbm.at[pl.ds(base, tpw)], rows_v)
        pltpu.sync_copy(rows_v, out_hbm.at[idx_v])

    return k(x, pos)


def _gather_inter(up, pos):
    tpw = S // NW
    sub = 32
    mesh = plsc.VectorSubcoreMesh(core_axis_name="c", subcore_axis_name="s")

    @functools.partial(
        pl.kernel, mesh=mesh,
        out_type=jax.ShapeDtypeStruct((S, I), jnp.float32),
        scratch_types=[pltpu.VMEM((sub,), jnp.int32),
                       pltpu.VMEM((sub, I), jnp.float32)],
    )
    def k(up_hbm, pos_hbm, out_hbm, idx_v, rows_v):
        wbase = _sc_wid() * tpw
        for j in range(tpw // sub):
            base = wbase + j * sub
            pltpu.sync_copy(pos_hbm.at[pl.ds(base, sub)], idx_v)
            pltpu.sync_copy(up_hbm.at[idx_v], rows_v)
            pltpu.sync_copy(rows_v, out_hbm.at[pl.ds(base, sub)])

    return k(up, pos)


def _gelu_exact(h):
    return 0.5 * h * (1.0 + lax.erf(h * 0.7071067811865476))


def _up_body(sc_ref, nc_ref, xs_ref, wi_ref, bi_ref, out_hbm, stg0, stg1, sems):
    e = pl.program_id(0)
    start = sc_ref[e]
    n = nc_ref[e]

    def dma(k, stg, idx):
        return pltpu.make_async_copy(
            stg, out_hbm.at[pl.ds((start + k) * C, C), :], sems.at[idx])

    def compute(k, stg):
        row = (start + k) * C
        xb = xs_ref[pl.ds(row, C), :].astype(jnp.bfloat16)
        h = jnp.dot(xb, wi_ref[0].astype(jnp.bfloat16),
                    preferred_element_type=jnp.float32)
        stg[...] = _gelu_exact(h + bi_ref[0])

    def pair(k2, carry):
        k = k2 * 2

        @pl.when(k2 > 0)
        def _():
            dma(0, stg0, 0).wait()

        compute(k, stg0)
        dma(k, stg0, 0).start()

        @pl.when(k + 1 < n)
        def _():
            @pl.when(k2 > 0)
            def _():
                dma(0, stg1, 1).wait()

            compute(k + 1, stg1)
            dma(k + 1, stg1, 1).start()

        return carry

    lax.fori_loop(0, (n + 1) // 2, pair, 0)

    @pl.when(n > 0)
    def _():
        dma(0, stg0, 0).wait()

    @pl.when(n > 1)
    def _():
        dma(0, stg1, 1).wait()


def _up(startc, nch, x_sorted, wi, bi):
    grid_spec = pltpu.PrefetchScalarGridSpec(
        num_scalar_prefetch=2,
        grid=(E,),
        in_specs=[
            pl.BlockSpec((P, H), lambda e, sc, nc: (0, 0)),
            pl.BlockSpec((1, H, I), lambda e, sc, nc: (e, 0, 0)),
            pl.BlockSpec((1, 1, I), lambda e, sc, nc: (e, 0, 0)),
        ],
        out_specs=pl.BlockSpec(memory_space=pl.ANY),
        scratch_shapes=[
            pltpu.VMEM((C, I), jnp.float32),
            pltpu.VMEM((C, I), jnp.float32),
            pltpu.SemaphoreType.DMA((2,)),
        ],
    )
    return pl.pallas_call(
        _up_body, grid_spec=grid_spec,
        out_shape=jax.ShapeDtypeStruct((P, I), jnp.float32),
    )(startc, nch, x_sorted, wi, bi.reshape(E, 1, I))


def _down_body(inter_ref, wo_ref, bo_ref, x_ref, g_ref, b_ref, y_ref):
    o = jnp.dot(inter_ref[...].astype(jnp.bfloat16),
                wo_ref[...].astype(jnp.bfloat16),
                preferred_element_type=jnp.float32)
    t = o + bo_ref[...] + x_ref[...]
    mu = jnp.mean(t, axis=1, keepdims=True)
    cen = t - mu
    var = jnp.mean(cen * cen, axis=1, keepdims=True)
    y_ref[...] = cen * lax.rsqrt(var + EPS) * g_ref[...] + b_ref[...]


def _down(inter, wo, bo, x, ln_g, ln_b):
    return pl.pallas_call(
        _down_body,
        grid=(S // DTB,),
        in_specs=[
            pl.BlockSpec((DTB, I), lambda t: (t, 0)),
            pl.BlockSpec((I, H), lambda t: (0, 0)),
            pl.BlockSpec((1, H), lambda t: (0, 0)),
            pl.BlockSpec((DTB, H), lambda t: (t, 0)),
            pl.BlockSpec((1, H), lambda t: (0, 0)),
            pl.BlockSpec((1, H), lambda t: (0, 0)),
        ],
        out_specs=pl.BlockSpec((DTB, H), lambda t: (t, 0)),
        out_shape=jax.ShapeDtypeStruct((S, H), jnp.float32),
    )(inter, wo, bo, x, ln_g, ln_b)


def kernel(hidden_states, w_router, wi, bi, wo, bo, ln_g, ln_b):
    b, s, h = hidden_states.shape
    x = hidden_states.reshape(s, h)
    pos2, startc2, nch2 = _routing(x, w_router)
    pos = pos2.reshape(s)
    x_sorted = _scatter_tokens(x, pos)
    up = _up(startc2.reshape(E), nch2.reshape(E), x_sorted, wi, bi)
    inter = _gather_inter(up, pos)
    y = _down(inter, wo, bo.reshape(1, h), x, ln_g.reshape(1, h), ln_b.reshape(1, h))
    return y.reshape(b, s, h)
